# Initial kernel scaffold; baseline (speedup 1.0000x reference)
import jax
import jax.numpy as jnp
from jax.experimental import pallas as pl


def kernel(pos, batch, W1, b1, g1, be1, W2, b2, g2, be2, W3, b3, g3, be3, W4, b4):
    raise NotImplementedError("write your pallas kernel here")


# speedup vs baseline: 2.0572x; 2.0572x over previous
import functools
import jax
import jax.numpy as jnp
from jax import lax
from jax.experimental import pallas as pl
from jax.experimental.pallas import tpu as pltpu

B = 32
NPTS = 1024
K = 20
EMB = 1024
OUT = 7
EPS = 1e-5
LANES = 128
BIG_I32 = 1 << 30


def _cloud_kernel(pos_ref, post_ref, wa_ref, ba_ref, wc_ref, w2_ref, b2_ref,
                  w3_ref, b3_ref, w4a_ref, w4b_ref, b4_ref, out_ref,
                  dist_ref, a_ref, c_ref, x1_ref):
    p = pos_ref[0]
    pt = post_ref[0]

    sq_r = jnp.sum(p * p, axis=1, keepdims=True)
    sq_c = jnp.sum(pt * pt, axis=0, keepdims=True)
    pp = jax.lax.dot(p, pt)
    dist_ref[...] = sq_r + sq_c - 2.0 * pp

    a_ref[...] = jax.lax.dot(p, wa_ref[...],
                             precision=jax.lax.Precision.HIGHEST) + ba_ref[...]
    c_ref[...] = jax.lax.dot(p, wc_ref[...],
                             precision=jax.lax.Precision.HIGHEST)
    x1_ref[...] = jnp.zeros((NPTS, 128), jnp.float32)

    iota = lax.broadcasted_iota(jnp.int32, (NPTS, NPTS), 1)

    def body(_, carry):
        d = dist_ref[...]
        m = jnp.min(d, axis=1, keepdims=True)
        loc = jnp.where(d == m, iota, BIG_I32)
        idx = jnp.min(loc, axis=1, keepdims=True)
        oh = iota == idx
        dist_ref[...] = jnp.where(oh, jnp.inf, d)
        nc = jax.lax.dot(oh.astype(jnp.float32), c_ref[...],
                         precision=jax.lax.Precision.HIGHEST)
        h1 = jnp.maximum(a_ref[...] + nc, 0.0)
        h2 = jnp.maximum(
            jax.lax.dot(h1, w2_ref[...],
                        precision=jax.lax.Precision.HIGHEST) + b2_ref[...], 0.0)
        x1_ref[...] = jnp.maximum(x1_ref[...], h2)
        return carry

    lax.fori_loop(0, K, body, 0)

    x = jnp.maximum(
        jax.lax.dot(x1_ref[...], w3_ref[...],
                    precision=jax.lax.Precision.HIGHEST) + b3_ref[...], 0.0)
    gmax = jnp.max(x, axis=0, keepdims=True)
    gmean = jnp.sum(x, axis=0, keepdims=True) * (1.0 / NPTS)
    o = (jax.lax.dot(gmax, w4a_ref[...], precision=jax.lax.Precision.HIGHEST)
         + jax.lax.dot(gmean, w4b_ref[...], precision=jax.lax.Precision.HIGHEST)
         + b4_ref[...])
    out_ref[0] = o


def kernel(pos, batch, W1, b1, g1, be1, W2, b2, g2, be2, W3, b3, g3, be3,
           W4, b4):
    del batch
    f32 = jnp.float32
    s1 = g1 / jnp.sqrt(1.0 + EPS)
    s2 = g2 / jnp.sqrt(1.0 + EPS)
    s3 = g3 / jnp.sqrt(1.0 + EPS)

    wa = (W1[:3] - W1[3:]) * s1[None, :]
    ba = (b1 * s1 + be1).reshape(1, 64)
    wc = W1[3:] * s1[None, :]
    w2 = W2 * s2[None, :]
    b2e = (b2 * s2 + be2).reshape(1, 128)
    w3 = W3 * s3[None, :]
    b3e = (b3 * s3 + be3).reshape(1, EMB)
    w4a = jnp.zeros((EMB, LANES), f32).at[:, :OUT].set(W4[:EMB])
    w4b = jnp.zeros((EMB, LANES), f32).at[:, :OUT].set(W4[EMB:])
    b4p = jnp.zeros((1, LANES), f32).at[0, :OUT].set(b4)

    pos3 = pos.reshape(B, NPTS, 3)
    post = jnp.swapaxes(pos3, 1, 2)

    rep = lambda shape: pl.BlockSpec(shape, lambda b: (0,) * len(shape))
    out = pl.pallas_call(
        _cloud_kernel,
        grid=(B,),
        in_specs=[
            pl.BlockSpec((1, NPTS, 3), lambda b: (b, 0, 0)),
            pl.BlockSpec((1, 3, NPTS), lambda b: (b, 0, 0)),
            rep((3, 64)), rep((1, 64)), rep((3, 64)),
            rep((64, 128)), rep((1, 128)),
            rep((128, EMB)), rep((1, EMB)),
            rep((EMB, LANES)), rep((EMB, LANES)), rep((1, LANES)),
        ],
        out_specs=pl.BlockSpec((1, 1, LANES), lambda b: (b, 0, 0)),
        out_shape=jax.ShapeDtypeStruct((B, 1, LANES), f32),
        scratch_shapes=[
            pltpu.VMEM((NPTS, NPTS), f32),
            pltpu.VMEM((NPTS, 64), f32),
            pltpu.VMEM((NPTS, 64), f32),
            pltpu.VMEM((NPTS, 128), f32),
        ],
    )(pos3, post, wa, ba, wc, w2, b2e, w3, b3e, w4a, w4b, b4p)
    return out.reshape(B, LANES)[:, :OUT]


# speedup vs baseline: 6.7143x; 3.2638x over previous
import functools
import jax
import jax.numpy as jnp
from jax import lax
from jax.experimental import pallas as pl
from jax.experimental.pallas import tpu as pltpu

B = 32
NPTS = 1024
K = 20
EMB = 1024
OUT = 7
EPS = 1e-5
LANES = 128
BIG_I32 = 1 << 30


def _cloud_kernel(pos_ref, post_ref, wa_ref, ba_ref, wc_ref, w2_ref, b2_ref,
                  w3_ref, b3_ref, w4a_ref, w4b_ref, b4_ref, out_ref,
                  dist_ref, a_ref, c_ref, x1_ref):
    p = pos_ref[0]
    pt = post_ref[0]

    sq_r = jnp.sum(p * p, axis=1, keepdims=True)
    sq_c = jnp.sum(pt * pt, axis=0, keepdims=True)
    pp = jax.lax.dot(p, pt)
    dist_ref[...] = sq_r + sq_c - 2.0 * pp

    a_ref[...] = jax.lax.dot(p, wa_ref[...],
                             precision=jax.lax.Precision.HIGHEST) + ba_ref[...]
    c_ref[...] = jax.lax.dot(p, wc_ref[...],
                             precision=jax.lax.Precision.HIGHEST)
    x1_ref[...] = jnp.zeros((NPTS, 128), jnp.float32)

    iota = lax.broadcasted_iota(jnp.int32, (NPTS, NPTS), 1)

    def body(_, carry):
        d = dist_ref[...]
        m = jnp.min(d, axis=1, keepdims=True)
        loc = jnp.where(d == m, iota, BIG_I32)
        idx = jnp.min(loc, axis=1, keepdims=True)
        oh = iota == idx
        dist_ref[...] = jnp.where(oh, jnp.inf, d)
        nc = jax.lax.dot(oh.astype(jnp.float32), c_ref[...])
        h1 = jnp.maximum(a_ref[...] + nc, 0.0)
        h2 = jnp.maximum(
            jax.lax.dot(h1, w2_ref[...]) + b2_ref[...], 0.0)
        x1_ref[...] = jnp.maximum(x1_ref[...], h2)
        return carry

    lax.fori_loop(0, K, body, 0)

    x = jnp.maximum(
        jax.lax.dot(x1_ref[...], w3_ref[...]) + b3_ref[...], 0.0)
    gmax = jnp.max(x, axis=0, keepdims=True)
    gmean = jnp.sum(x, axis=0, keepdims=True) * (1.0 / NPTS)
    o = (jax.lax.dot(gmax, w4a_ref[...]) + jax.lax.dot(gmean, w4b_ref[...])
         + b4_ref[...])
    out_ref[0] = o


def kernel(pos, batch, W1, b1, g1, be1, W2, b2, g2, be2, W3, b3, g3, be3,
           W4, b4):
    del batch
    f32 = jnp.float32
    s1 = g1 / jnp.sqrt(1.0 + EPS)
    s2 = g2 / jnp.sqrt(1.0 + EPS)
    s3 = g3 / jnp.sqrt(1.0 + EPS)

    wa = (W1[:3] - W1[3:]) * s1[None, :]
    ba = (b1 * s1 + be1).reshape(1, 64)
    wc = W1[3:] * s1[None, :]
    w2 = W2 * s2[None, :]
    b2e = (b2 * s2 + be2).reshape(1, 128)
    w3 = W3 * s3[None, :]
    b3e = (b3 * s3 + be3).reshape(1, EMB)
    w4a = jnp.zeros((EMB, LANES), f32).at[:, :OUT].set(W4[:EMB])
    w4b = jnp.zeros((EMB, LANES), f32).at[:, :OUT].set(W4[EMB:])
    b4p = jnp.zeros((1, LANES), f32).at[0, :OUT].set(b4)

    pos3 = pos.reshape(B, NPTS, 3)
    post = jnp.swapaxes(pos3, 1, 2)

    rep = lambda shape: pl.BlockSpec(shape, lambda b: (0,) * len(shape))
    out = pl.pallas_call(
        _cloud_kernel,
        grid=(B,),
        in_specs=[
            pl.BlockSpec((1, NPTS, 3), lambda b: (b, 0, 0)),
            pl.BlockSpec((1, 3, NPTS), lambda b: (b, 0, 0)),
            rep((3, 64)), rep((1, 64)), rep((3, 64)),
            rep((64, 128)), rep((1, 128)),
            rep((128, EMB)), rep((1, EMB)),
            rep((EMB, LANES)), rep((EMB, LANES)), rep((1, LANES)),
        ],
        out_specs=pl.BlockSpec((1, 1, LANES), lambda b: (b, 0, 0)),
        out_shape=jax.ShapeDtypeStruct((B, 1, LANES), f32),
        scratch_shapes=[
            pltpu.VMEM((NPTS, NPTS), f32),
            pltpu.VMEM((NPTS, 64), f32),
            pltpu.VMEM((NPTS, 64), f32),
            pltpu.VMEM((NPTS, 128), f32),
        ],
    )(pos3, post, wa, ba, wc, w2, b2e, w3, b3e, w4a, w4b, b4p)
    return out.reshape(B, LANES)[:, :OUT]


# speedup vs baseline: 8.1075x; 1.2075x over previous
import functools
import jax
import jax.numpy as jnp
from jax import lax
from jax.experimental import pallas as pl
from jax.experimental.pallas import tpu as pltpu

B = 32
NPTS = 1024
K = 20
EMB = 1024
OUT = 7
EPS = 1e-5
LANES = 128
BIG_F = 1e9


def _cloud_kernel(pos_ref, post_ref, wa_ref, ba_ref, wc_ref, w2_ref, b2_ref,
                  w3_ref, b3_ref, w4a_ref, w4b_ref, b4_ref, out_ref,
                  dist_ref, a_ref, c_ref, x1_ref):
    p = pos_ref[0]
    pt = post_ref[0]

    sq_r = jnp.sum(p * p, axis=1, keepdims=True)
    sq_c = jnp.sum(pt * pt, axis=0, keepdims=True)
    pp = jax.lax.dot(p, pt)
    dist_ref[...] = sq_r + sq_c - 2.0 * pp

    a_ref[...] = jax.lax.dot(p, wa_ref[...],
                             precision=jax.lax.Precision.HIGHEST) + ba_ref[...]
    c_ref[...] = jax.lax.dot(p, wc_ref[...],
                             precision=jax.lax.Precision.HIGHEST)
    x1_ref[...] = jnp.zeros((NPTS, 128), jnp.float32)

    iota = lax.broadcasted_iota(jnp.int32, (NPTS, NPTS), 1).astype(jnp.float32)

    def body(_, carry):
        d = dist_ref[...]
        m = jnp.min(d, axis=1, keepdims=True)
        loc = jnp.where(d == m, iota, jnp.float32(BIG_F))
        idx = jnp.min(loc, axis=1, keepdims=True)
        oh = iota == idx
        dist_ref[...] = jnp.where(oh, jnp.inf, d)
        nc = jax.lax.dot(oh.astype(jnp.float32), c_ref[...])
        h1 = jnp.maximum(a_ref[...] + nc, 0.0)
        h2 = jnp.maximum(
            jax.lax.dot(h1, w2_ref[...]) + b2_ref[...], 0.0)
        x1_ref[...] = jnp.maximum(x1_ref[...], h2)
        return carry

    lax.fori_loop(0, K, body, 0)

    x = jnp.maximum(
        jax.lax.dot(x1_ref[...], w3_ref[...]) + b3_ref[...], 0.0)
    gmax = jnp.max(x, axis=0, keepdims=True)
    gmean = jnp.sum(x, axis=0, keepdims=True) * (1.0 / NPTS)
    o = (jax.lax.dot(gmax, w4a_ref[...]) + jax.lax.dot(gmean, w4b_ref[...])
         + b4_ref[...])
    out_ref[0] = o


def kernel(pos, batch, W1, b1, g1, be1, W2, b2, g2, be2, W3, b3, g3, be3,
           W4, b4):
    del batch
    f32 = jnp.float32
    s1 = g1 / jnp.sqrt(1.0 + EPS)
    s2 = g2 / jnp.sqrt(1.0 + EPS)
    s3 = g3 / jnp.sqrt(1.0 + EPS)

    wa = (W1[:3] - W1[3:]) * s1[None, :]
    ba = (b1 * s1 + be1).reshape(1, 64)
    wc = W1[3:] * s1[None, :]
    w2 = W2 * s2[None, :]
    b2e = (b2 * s2 + be2).reshape(1, 128)
    w3 = W3 * s3[None, :]
    b3e = (b3 * s3 + be3).reshape(1, EMB)
    w4a = jnp.zeros((EMB, LANES), f32).at[:, :OUT].set(W4[:EMB])
    w4b = jnp.zeros((EMB, LANES), f32).at[:, :OUT].set(W4[EMB:])
    b4p = jnp.zeros((1, LANES), f32).at[0, :OUT].set(b4)

    pos3 = pos.reshape(B, NPTS, 3)
    post = jnp.swapaxes(pos3, 1, 2)

    rep = lambda shape: pl.BlockSpec(shape, lambda b: (0,) * len(shape))
    out = pl.pallas_call(
        _cloud_kernel,
        grid=(B,),
        in_specs=[
            pl.BlockSpec((1, NPTS, 3), lambda b: (b, 0, 0)),
            pl.BlockSpec((1, 3, NPTS), lambda b: (b, 0, 0)),
            rep((3, 64)), rep((1, 64)), rep((3, 64)),
            rep((64, 128)), rep((1, 128)),
            rep((128, EMB)), rep((1, EMB)),
            rep((EMB, LANES)), rep((EMB, LANES)), rep((1, LANES)),
        ],
        out_specs=pl.BlockSpec((1, 1, LANES), lambda b: (b, 0, 0)),
        out_shape=jax.ShapeDtypeStruct((B, 1, LANES), f32),
        scratch_shapes=[
            pltpu.VMEM((NPTS, NPTS), f32),
            pltpu.VMEM((NPTS, 64), f32),
            pltpu.VMEM((NPTS, 64), f32),
            pltpu.VMEM((NPTS, 128), f32),
        ],
    )(pos3, post, wa, ba, wc, w2, b2e, w3, b3e, w4a, w4b, b4p)
    return out.reshape(B, LANES)[:, :OUT]
